# baseline (speedup 1.0000x reference)
import functools

import jax
import jax.numpy as jnp
from jax import lax
from jax.experimental import pallas as pl
from jax.experimental.pallas import tpu as pltpu
from jax.experimental.pallas import tpu_sc as plsc

_VOCAB = 1000000
_EMB = 64
_B = 4096
_L = 200
_N = _B * _L
_SCALE = 8.0

_NC = 2
_NS = 16
_NW = _NC * _NS
_CH = 128
_GRP = _N // (_NW * _CH)


def _emb_body(tokens_hbm, table_hbm, out_hbm, idx_v, rows_v, gsem0, gsem1):
    wid = lax.axis_index("s") * _NC + lax.axis_index("c")
    g0 = wid * _GRP

    pltpu.sync_copy(tokens_hbm.at[pl.ds(g0, _GRP)], idx_v)

    gsems = (gsem0, gsem1)

    def start_gather(g, b):
        pltpu.async_copy(table_hbm.at[idx_v.at[g]], rows_v.at[b], gsems[b])

    def wait_gather(g, b):
        pltpu.make_async_copy(
            table_hbm.at[idx_v.at[g]], rows_v.at[b], gsems[b]
        ).wait()

    start_gather(0, 0)
    start_gather(1, 1)

    def pair_body(i, carry):
        for b in range(2):
            g = 2 * i + b
            wait_gather(g, b)

            def scale_row(r, c):
                for j in range(_EMB // 16):
                    sl = pl.ds(j * 16, 16)
                    rows_v[b, r, sl] = rows_v[b, r, sl] * _SCALE
                return c

            lax.fori_loop(0, _CH, scale_row, 0, unroll=4)

            pltpu.sync_copy(
                rows_v.at[b], out_hbm.at[pl.ds((g0 + g) * _CH, _CH)]
            )

            @pl.when(g + 2 < _GRP)
            def _():
                start_gather(g + 2, b)

        return carry

    lax.fori_loop(0, _GRP // 2, pair_body, 0)


@jax.jit
def _embed(tokens2d, table):
    run = functools.partial(
        pl.kernel,
        mesh=plsc.VectorSubcoreMesh(core_axis_name="c", subcore_axis_name="s"),
        out_type=jax.ShapeDtypeStruct((_N, _EMB), jnp.float32),
        scratch_types=[
            pltpu.VMEM((_GRP, _CH), jnp.int32),
            pltpu.VMEM((2, _CH, _EMB), jnp.float32),
            pltpu.SemaphoreType.DMA,
            pltpu.SemaphoreType.DMA,
        ],
        compiler_params=pltpu.CompilerParams(use_tc_tiling_on_sc=False),
    )(_emb_body)
    return run(tokens2d, table)


def kernel(tokens, table):
    tokens2d = tokens.reshape(_N // _CH, _CH)
    out = _embed(tokens2d, table)
    return out.reshape(_B, _L, _EMB)
